# baseline (speedup 1.0000x reference)
import functools

import jax
import jax.numpy as jnp
from jax import lax
from jax.experimental import pallas as pl
from jax.experimental.pallas import tpu as pltpu
from jax.experimental.pallas import tpu_sc as plsc

NUM_ENTITIES = 1_000_000
DIM = 64
NI, NJ = 16384, 26
B = NI * NJ
NC, NS = 2, 16
NW = NC * NS
K = 256
IB = NI // K
M = (NJ * IB) // NW
BPW = M * K

_mesh = plsc.VectorSubcoreMesh(core_axis_name="c", subcore_axis_name="s")


@functools.partial(
    pl.kernel,
    mesh=_mesh,
    out_type=jax.ShapeDtypeStruct((NJ, DIM, NI), jnp.float32),
    compiler_params=pltpu.CompilerParams(needs_layout_passes=False),
    scratch_types=[
        pltpu.VMEM((BPW,), jnp.int32),
        pltpu.VMEM((BPW,), jnp.int32),
        pltpu.VMEM((2, K, 2 * DIM), jnp.float32),
        pltpu.VMEM((2, DIM, K), jnp.float32),
        pltpu.SemaphoreType.DMA,
        pltpu.SemaphoreType.DMA,
        pltpu.SemaphoreType.DMA,
        pltpu.SemaphoreType.DMA,
    ],
)
def _embed32(ids_hbm, w128_hbm, out_hbm, idx_v, ihi_v, g_v, t_v,
             gsem0, gsem1, ssem0, ssem1):
    wid = lax.axis_index("s") * NC + lax.axis_index("c")
    base = wid * BPW

    pltpu.sync_copy(ids_hbm.at[pl.ds(base, BPW)], idx_v)

    def _pre(k, _):
        sl = pl.ds(k * 16, 16)
        ihi_v[sl] = lax.shift_right_logical(idx_v[sl], 1)
        return _
    lax.fori_loop(0, BPW // 16, _pre, None)

    gsems = (gsem0, gsem1)
    ssems = (ssem0, ssem1)

    def _fire_gather(m, s):
        pltpu.async_copy(
            w128_hbm.at[ihi_v.at[pl.ds(m * K, K)]], g_v.at[s], gsems[s])

    def _wait_gather(s):
        pltpu.make_async_copy(w128_hbm.at[ihi_v.at[pl.ds(0, K)]],
                              g_v.at[s], gsems[s]).wait()

    def _out_slice(m):
        gm = wid * M + m
        j = gm // IB
        i0 = (gm % IB) * K
        return out_hbm.at[j, :, pl.ds(i0, K)]

    def _fire_store(m, s):
        pltpu.async_copy(t_v.at[s], _out_slice(m), ssems[s])

    def _wait_store(s):
        pltpu.make_async_copy(t_v.at[s], _out_slice(0), ssems[s]).wait()

    _fire_gather(0, 0)
    _fire_gather(1, 1)

    def _item(i, _):
        for s in (0, 1):
            m = 2 * i + s
            _wait_gather(s)

            @pl.when(m >= 2)
            def _():
                _wait_store(s)

            def _grp(g, _):
                sl = pl.ds(m * K + g * 16, 16)
                h64 = lax.shift_left(
                    lax.bitwise_and(idx_v[sl], jnp.int32(1)), jnp.int32(6))
                rvec = lax.iota(jnp.int32, 16) + g * 16
                for d in range(DIM):
                    vals = plsc.load_gather(g_v.at[s], [rvec, h64 + d])
                    t_v[s, d, pl.ds(g * 16, 16)] = vals
                return _
            lax.fori_loop(0, K // 16, _grp, None)

            _fire_store(m, s)

            @pl.when(m + 2 < M)
            def _():
                _fire_gather(m + 2, s)
        return _
    lax.fori_loop(0, M // 2, _item, None)

    _wait_store(0)
    _wait_store(1)


def kernel(ids, weight):
    ids_lin = jnp.transpose(ids).reshape(-1)
    w128 = weight.reshape(NUM_ENTITIES // 2, 2 * DIM)
    out_t = _embed32(ids_lin, w128)
    return jnp.transpose(out_t, (2, 0, 1))
